# Initial kernel scaffold; baseline (speedup 1.0000x reference)
import jax
import jax.numpy as jnp
from jax.experimental import pallas as pl


def kernel(mUser, m_item, id_embds, A, l0_w1, l0_b1, l0_w2, l0_b2, l0_w3h, l0_w3x, l0_b3, l1_w1, l1_b1, l1_w2, l1_b2, l1_w3h, l1_w3x, l1_b3, l2_w1, l2_b1, l2_w2, l2_b2, l2_w3h, l2_w3x, l2_b3):
    raise NotImplementedError("write your pallas kernel here")


# speedup vs baseline: 2.0577x; 2.0577x over previous
import jax
import jax.numpy as jnp
from jax.experimental import pallas as pl
from jax.experimental.pallas import tpu as pltpu

NEG_SLOPE = 0.01
EPS = 1e-12
TILE = 512


def _leaky(v):
    return jnp.where(v >= 0, v, NEG_SLOPE * v)


def _gcn_body(x0_ref, id_ref, a_ref, w1_ref, b1_ref, w2_ref, b2_ref,
              w3h_ref, w3x_ref, b3_ref, o_ref,
              abf_ref, y1_ref, xh_ref, x_ref, acc_ref):
    l = pl.program_id(0)
    j = pl.program_id(1)
    k = pl.program_id(2)
    nt = pl.num_programs(2)
    rows = pl.ds(k * TILE, TILE)

    @pl.when(j == 0)
    def _prepass():
        x0 = x0_ref[...]
        nrm2 = jnp.sum(x0 * x0, axis=-1, keepdims=True)
        x0n = (x0 * jax.lax.rsqrt(jnp.maximum(nrm2, EPS * EPS))
               ).astype(jnp.bfloat16)
        xb = jnp.where(l == 0, x0n, x_ref[rows, :])

        y1 = jnp.dot(xb, w1_ref[0], preferred_element_type=jnp.float32)
        y1_ref[rows, :] = (y1 + b1_ref[0]).astype(jnp.bfloat16)

        y2 = jnp.dot(xb, w2_ref[0], preferred_element_type=jnp.float32)
        xh_ref[rows, :] = (_leaky(y2 + b2_ref[0]) + id_ref[rows, :]
                           ).astype(jnp.bfloat16)

    @pl.when(j > 0)
    def _aggregate():
        i = j - 1
        irows = pl.ds(i * TILE, TILE)
        kcols = pl.ds(k * TILE, TILE)

        @pl.when(k == 0)
        def _():
            acc_ref[...] = jnp.zeros_like(acc_ref)

        @pl.when(l == 0)
        def _():
            abf_ref[irows, kcols] = a_ref[...].astype(jnp.bfloat16)

        acc_ref[...] += jnp.dot(abf_ref[irows, kcols], y1_ref[rows, :],
                                preferred_element_type=jnp.float32)

        @pl.when(k == nt - 1)
        def _finish():
            h = _leaky(acc_ref[...]).astype(jnp.bfloat16)
            o = (jnp.dot(h, w3h_ref[0], preferred_element_type=jnp.float32)
                 + jnp.dot(xh_ref[irows, :], w3x_ref[0],
                           preferred_element_type=jnp.float32)
                 + b3_ref[0])
            o = _leaky(o)

            @pl.when(l < 2)
            def _():
                x_ref[irows, :] = o.astype(jnp.bfloat16)

            @pl.when(l == 2)
            def _():
                o_ref[...] = o


def kernel(mUser, m_item, id_embds, A,
           l0_w1, l0_b1, l0_w2, l0_b2, l0_w3h, l0_w3x, l0_b3,
           l1_w1, l1_b1, l1_w2, l1_b2, l1_w3h, l1_w3x, l1_b3,
           l2_w1, l2_b1, l2_w2, l2_b2, l2_w3h, l2_w3x, l2_b3):
    x0 = jnp.concatenate([mUser, m_item], axis=0)
    n, d = x0.shape
    e = id_embds.shape[1]
    nt = n // TILE

    bf = jnp.bfloat16
    w1s = jnp.stack([l0_w1, l1_w1, l2_w1]).astype(bf)
    w2s = jnp.stack([l0_w2, l1_w2, l2_w2]).astype(bf)
    w3hs = jnp.stack([l0_w3h, l1_w3h, l2_w3h]).astype(bf)
    w3xs = jnp.stack([l0_w3x, l1_w3x, l2_w3x]).astype(bf)
    b1s = jnp.stack([l0_b1, l1_b1, l2_b1])
    b2s = jnp.stack([l0_b2, l1_b2, l2_b2])
    b3s = jnp.stack([l0_b3, l1_b3, l2_b3])

    zero2 = lambda l, j, k: (0, 0)
    layer3 = lambda l, j, k: (l, 0, 0)
    a_map = lambda l, j, k: (jnp.where((l == 0) & (j > 0), j - 1, 0),
                             jnp.where((l == 0) & (j > 0), k, 0))
    x0_map = lambda l, j, k: (jnp.where((l == 0) & (j == 0), k, 0), 0)
    o_map = lambda l, j, k: (jnp.where((l == 2) & (j > 0), j - 1, 0), 0)

    return pl.pallas_call(
        _gcn_body,
        out_shape=jax.ShapeDtypeStruct((n, e), jnp.float32),
        grid=(3, nt + 1, nt),
        in_specs=[
            pl.BlockSpec((TILE, d), x0_map),
            pl.BlockSpec((n, e), zero2),
            pl.BlockSpec((TILE, TILE), a_map),
            pl.BlockSpec((1, d, d), layer3),
            pl.BlockSpec((1, 1, d), layer3),
            pl.BlockSpec((1, d, e), layer3),
            pl.BlockSpec((1, 1, e), layer3),
            pl.BlockSpec((1, d, e), layer3),
            pl.BlockSpec((1, e, e), layer3),
            pl.BlockSpec((1, 1, e), layer3),
        ],
        out_specs=pl.BlockSpec((TILE, e), o_map),
        scratch_shapes=[
            pltpu.VMEM((n, n), jnp.bfloat16),
            pltpu.VMEM((n, d), jnp.bfloat16),
            pltpu.VMEM((n, e), jnp.bfloat16),
            pltpu.VMEM((n, e), jnp.bfloat16),
            pltpu.VMEM((TILE, d), jnp.float32),
        ],
        compiler_params=pltpu.CompilerParams(
            dimension_semantics=("arbitrary", "arbitrary", "arbitrary"),
            vmem_limit_bytes=60 * 1024 * 1024),
    )(x0, id_embds, A, w1s, b1s, w2s, b2s, w3hs, w3xs, b3s)


# speedup vs baseline: 3.3234x; 1.6151x over previous
import functools

import jax
import jax.numpy as jnp
from jax.experimental import pallas as pl
from jax.experimental.pallas import tpu as pltpu

NEG_SLOPE = 0.01
EPS = 1e-12
TILE = 512
KBLK = 1024


def _leaky(v):
    return jnp.where(v >= 0, v, NEG_SLOPE * v)


def _gcn_body(x0_ref, id_ref, a_ref, w1_ref, b1_ref, w2_ref, b2_ref,
              w3h_ref, w3x_ref, b3_ref, o_ref,
              abf_ref, y1_ref, xh_ref, x_ref, acc_ref, *, nk, bnd):
    b0, b1, b2, b3, b4, b5 = bnd
    s = pl.program_id(0)
    is_pre = (s < b0) | ((s >= b1) & (s < b2)) | ((s >= b3) & (s < b4))

    @pl.when(is_pre)
    def _prepass():
        r = jnp.where(s < b0, s, jnp.where(s < b2, s - b1, s - b3))
        rows = pl.ds(r * TILE, TILE)
        x0 = x0_ref[...]
        nrm2 = jnp.sum(x0 * x0, axis=-1, keepdims=True)
        x0n = (x0 * jax.lax.rsqrt(jnp.maximum(nrm2, EPS * EPS))
               ).astype(jnp.bfloat16)
        xb = jnp.where(s < b0, x0n, x_ref[rows, :])

        y1 = jnp.dot(xb, w1_ref[0], preferred_element_type=jnp.float32)
        y1_ref[rows, :] = (y1 + b1_ref[0]).astype(jnp.bfloat16)

        y2 = jnp.dot(xb, w2_ref[0], preferred_element_type=jnp.float32)
        xh_ref[rows, :] = (_leaky(y2 + b2_ref[0]) + id_ref[rows, :]
                           ).astype(jnp.bfloat16)

    @pl.when((s >= b0) & (s < b1))
    def _l0_aggregate():
        t = s - b0
        i = t // nk
        kk = t % nk
        irows = pl.ds(i * TILE, TILE)

        @pl.when(kk == 0)
        def _():
            acc_ref[...] = jnp.zeros_like(acc_ref)

        ab = a_ref[...].astype(jnp.bfloat16)
        abf_ref[irows, pl.ds(kk * KBLK, KBLK)] = ab
        acc_ref[...] += jnp.dot(ab, y1_ref[pl.ds(kk * KBLK, KBLK), :],
                                preferred_element_type=jnp.float32)

        @pl.when(kk == nk - 1)
        def _():
            h = _leaky(acc_ref[...]).astype(jnp.bfloat16)
            o = (jnp.dot(h, w3h_ref[0], preferred_element_type=jnp.float32)
                 + jnp.dot(xh_ref[irows, :], w3x_ref[0],
                           preferred_element_type=jnp.float32)
                 + b3_ref[0])
            x_ref[irows, :] = _leaky(o).astype(jnp.bfloat16)

    @pl.when(((s >= b2) & (s < b3)) | (s >= b4))
    def _l12_aggregate():
        i = jnp.where(s < b3, s - b2, s - b4)
        irows = pl.ds(i * TILE, TILE)
        acc = jnp.dot(abf_ref[irows, :], y1_ref[...],
                      preferred_element_type=jnp.float32)
        h = _leaky(acc).astype(jnp.bfloat16)
        o = (jnp.dot(h, w3h_ref[0], preferred_element_type=jnp.float32)
             + jnp.dot(xh_ref[irows, :], w3x_ref[0],
                       preferred_element_type=jnp.float32)
             + b3_ref[0])
        o = _leaky(o)

        @pl.when(s < b3)
        def _():
            x_ref[irows, :] = o.astype(jnp.bfloat16)

        @pl.when(s >= b4)
        def _():
            o_ref[...] = o


def kernel(mUser, m_item, id_embds, A,
           l0_w1, l0_b1, l0_w2, l0_b2, l0_w3h, l0_w3x, l0_b3,
           l1_w1, l1_b1, l1_w2, l1_b2, l1_w3h, l1_w3x, l1_b3,
           l2_w1, l2_b1, l2_w2, l2_b2, l2_w3h, l2_w3x, l2_b3):
    x0 = jnp.concatenate([mUser, m_item], axis=0)
    n, d = x0.shape
    e = id_embds.shape[1]
    nt = n // TILE
    nk = n // KBLK
    b0 = nt
    b1 = b0 + nt * nk
    b2 = b1 + nt
    b3 = b2 + nt
    b4 = b3 + nt
    b5 = b4 + nt
    bnd = (b0, b1, b2, b3, b4, b5)

    bf = jnp.bfloat16
    w1s = jnp.stack([l0_w1, l1_w1, l2_w1]).astype(bf)
    w2s = jnp.stack([l0_w2, l1_w2, l2_w2]).astype(bf)
    w3hs = jnp.stack([l0_w3h, l1_w3h, l2_w3h]).astype(bf)
    w3xs = jnp.stack([l0_w3x, l1_w3x, l2_w3x]).astype(bf)
    b1s = jnp.stack([l0_b1, l1_b1, l2_b1])
    b2s = jnp.stack([l0_b2, l1_b2, l2_b2])
    b3s = jnp.stack([l0_b3, l1_b3, l2_b3])

    zero2 = lambda s: (0, 0)
    wmap = lambda s: (jnp.where(s < b1, 0, jnp.where(s < b3, 1, 2)), 0, 0)
    a_map = lambda s: (
        jnp.where(s < b0, 0, jnp.where(s >= b1, nt - 1, (s - b0) // nk)),
        jnp.where(s < b0, 0, jnp.where(s >= b1, nk - 1, (s - b0) % nk)))
    x0_map = lambda s: (jnp.where(s < b0, s, 0), 0)
    o_map = lambda s: (jnp.where(s >= b4, s - b4, 0), 0)

    body = functools.partial(_gcn_body, nk=nk, bnd=bnd)
    return pl.pallas_call(
        body,
        out_shape=jax.ShapeDtypeStruct((n, e), jnp.float32),
        grid=(b5,),
        in_specs=[
            pl.BlockSpec((TILE, d), x0_map),
            pl.BlockSpec((n, e), zero2),
            pl.BlockSpec((TILE, KBLK), a_map),
            pl.BlockSpec((1, d, d), wmap),
            pl.BlockSpec((1, 1, d), wmap),
            pl.BlockSpec((1, d, e), wmap),
            pl.BlockSpec((1, 1, e), wmap),
            pl.BlockSpec((1, d, e), wmap),
            pl.BlockSpec((1, e, e), wmap),
            pl.BlockSpec((1, 1, e), wmap),
        ],
        out_specs=pl.BlockSpec((TILE, e), o_map),
        scratch_shapes=[
            pltpu.VMEM((n, n), jnp.bfloat16),
            pltpu.VMEM((n, d), jnp.bfloat16),
            pltpu.VMEM((n, e), jnp.bfloat16),
            pltpu.VMEM((n, e), jnp.bfloat16),
            pltpu.VMEM((TILE, d), jnp.float32),
        ],
        compiler_params=pltpu.CompilerParams(
            dimension_semantics=("arbitrary",),
            vmem_limit_bytes=60 * 1024 * 1024),
    )(x0, id_embds, A, w1s, b1s, w2s, b2s, w3hs, w3xs, b3s)


# speedup vs baseline: 3.7763x; 1.1363x over previous
import functools

import jax
import jax.numpy as jnp
from jax.experimental import pallas as pl
from jax.experimental.pallas import tpu as pltpu

NEG_SLOPE = 0.01
EPS = 1e-12
TILE = 512
KBLK = 1024


def _leaky(v):
    return jnp.where(v >= 0, v, NEG_SLOPE * v)


def _gcn_body(*refs, nk, nut, bnd):
    b0, b1, b2, b3, b4, b5 = bnd
    mu_ref, mi_ref, id_ref = refs[:3]
    a_refs = refs[3:3 + nk]
    w = refs[3 + nk:3 + nk + 21]
    o_ref = refs[3 + nk + 21]
    abf_ref, y1_ref, xh_ref, x_ref, acc_ref = refs[3 + nk + 22:]

    s = pl.program_id(0)
    bf = jnp.bfloat16

    def sel3(i0):
        return jnp.where(s < b1, w[i0][...],
                         jnp.where(s < b3, w[i0 + 7][...], w[i0 + 14][...]))

    is_pre = (s < b0) | ((s >= b1) & (s < b2)) | ((s >= b3) & (s < b4))

    @pl.when(is_pre)
    def _prepass():
        r = jnp.where(s < b0, s, jnp.where(s < b2, s - b1, s - b3))
        rows = pl.ds(r * TILE, TILE)
        x0 = jnp.where(r < nut, mu_ref[...], mi_ref[...])
        nrm2 = jnp.sum(x0 * x0, axis=-1, keepdims=True)
        x0n = (x0 * jax.lax.rsqrt(jnp.maximum(nrm2, EPS * EPS))).astype(bf)
        xb = jnp.where(s < b0, x0n, x_ref[rows, :])

        y1 = jnp.dot(xb, sel3(0).astype(bf), preferred_element_type=jnp.float32)
        y1_ref[rows, :] = (y1 + sel3(1)).astype(bf)

        y2 = jnp.dot(xb, sel3(2).astype(bf), preferred_element_type=jnp.float32)
        xh_ref[rows, :] = (_leaky(y2 + sel3(3)) + id_ref[rows, :]).astype(bf)

    @pl.when((s >= b0) & (s < b1))
    def _l0_aggregate():
        t = s - b0
        i = t // nk
        kk = t % nk
        irows = pl.ds(i * TILE, TILE)

        @pl.when(kk == 0)
        def _():
            acc_ref[...] = jnp.zeros_like(acc_ref)

        for j in range(nk):
            @pl.when(kk == j)
            def _(j=j):
                ab = a_refs[j][...].astype(bf)
                abf_ref[irows, j * KBLK:(j + 1) * KBLK] = ab
                acc_ref[...] += jnp.dot(
                    ab, y1_ref[j * KBLK:(j + 1) * KBLK, :],
                    preferred_element_type=jnp.float32)

        @pl.when(kk == nk - 1)
        def _():
            h = _leaky(acc_ref[...]).astype(bf)
            o = (jnp.dot(h, sel3(4).astype(bf),
                         preferred_element_type=jnp.float32)
                 + jnp.dot(xh_ref[irows, :], sel3(5).astype(bf),
                           preferred_element_type=jnp.float32)
                 + sel3(6))
            x_ref[irows, :] = _leaky(o).astype(bf)

    @pl.when(((s >= b2) & (s < b3)) | (s >= b4))
    def _l12_aggregate():
        i = jnp.where(s < b3, s - b2, s - b4)
        irows = pl.ds(i * TILE, TILE)
        acc = jnp.dot(abf_ref[irows, :], y1_ref[...],
                      preferred_element_type=jnp.float32)
        h = _leaky(acc).astype(bf)
        o = (jnp.dot(h, sel3(4).astype(bf), preferred_element_type=jnp.float32)
             + jnp.dot(xh_ref[irows, :], sel3(5).astype(bf),
                       preferred_element_type=jnp.float32)
             + sel3(6))
        o = _leaky(o)

        @pl.when(s < b3)
        def _():
            x_ref[irows, :] = o.astype(bf)

        @pl.when(s >= b4)
        def _():
            o_ref[...] = o


def kernel(mUser, m_item, id_embds, A,
           l0_w1, l0_b1, l0_w2, l0_b2, l0_w3h, l0_w3x, l0_b3,
           l1_w1, l1_b1, l1_w2, l1_b2, l1_w3h, l1_w3x, l1_b3,
           l2_w1, l2_b1, l2_w2, l2_b2, l2_w3h, l2_w3x, l2_b3):
    nu, d = mUser.shape
    ni = m_item.shape[0]
    n = nu + ni
    e = id_embds.shape[1]
    nt = n // TILE
    nut = nu // TILE
    nit = ni // TILE
    nk = n // KBLK
    b0 = nt
    b1 = b0 + nt * nk
    b2 = b1 + nt
    b3 = b2 + nt
    b4 = b3 + nt
    b5 = b4 + nt
    bnd = (b0, b1, b2, b3, b4, b5)

    zero2 = lambda s: (0, 0)
    mu_map = lambda s: (jnp.clip(s, 0, nut - 1), 0)
    mi_map = lambda s: (jnp.clip(s - nut, 0, nit - 1), 0)
    a_maps = [functools.partial(
        lambda s, j: (jnp.clip((s - b0 - j + nk - 1) // nk, 0, nt - 1), j),
        j=j) for j in range(nk)]
    o_map = lambda s: (jnp.where(s >= b4, s - b4, 0), 0)

    weights = [l0_w1, l0_b1, l0_w2, l0_b2, l0_w3h, l0_w3x, l0_b3,
               l1_w1, l1_b1, l1_w2, l1_b2, l1_w3h, l1_w3x, l1_b3,
               l2_w1, l2_b1, l2_w2, l2_b2, l2_w3h, l2_w3x, l2_b3]

    body = functools.partial(_gcn_body, nk=nk, nut=nut, bnd=bnd)
    return pl.pallas_call(
        body,
        out_shape=jax.ShapeDtypeStruct((n, e), jnp.float32),
        grid=(b5,),
        in_specs=[
            pl.BlockSpec((TILE, d), mu_map),
            pl.BlockSpec((TILE, d), mi_map),
            pl.BlockSpec((n, e), zero2),
        ] + [
            pl.BlockSpec((TILE, KBLK), m) for m in a_maps
        ] + [
            pl.BlockSpec(wa.shape, zero2) for wa in weights
        ],
        out_specs=pl.BlockSpec((TILE, e), o_map),
        scratch_shapes=[
            pltpu.VMEM((n, n), jnp.bfloat16),
            pltpu.VMEM((n, d), jnp.bfloat16),
            pltpu.VMEM((n, e), jnp.bfloat16),
            pltpu.VMEM((n, e), jnp.bfloat16),
            pltpu.VMEM((TILE, d), jnp.float32),
        ],
        compiler_params=pltpu.CompilerParams(
            dimension_semantics=("arbitrary",),
            vmem_limit_bytes=60 * 1024 * 1024),
    )(mUser, m_item, id_embds, *([A] * nk), *weights)


# speedup vs baseline: 4.3925x; 1.1632x over previous
import functools

import jax
import jax.numpy as jnp
from jax.experimental import pallas as pl
from jax.experimental.pallas import tpu as pltpu

NEG_SLOPE = 0.01
EPS = 1e-12
TILE = 512
KROW = 256


def _leaky(v):
    return jnp.where(v >= 0, v, NEG_SLOPE * v)


def _gcn_body(*refs, nut, bnd):
    b0, b1, b2, b3, b4, b5 = bnd
    mu_ref, mi_ref, id_ref, a_ref = refs[:4]
    w = refs[4:4 + 21]
    o_ref = refs[4 + 21]
    abf_ref, y1_ref, xh_ref, x_ref = refs[4 + 22:]

    s = pl.program_id(0)
    bf = jnp.bfloat16

    def sel3(i0):
        return jnp.where(s < b1, w[i0][...],
                         jnp.where(s < b3, w[i0 + 7][...], w[i0 + 14][...]))

    def fuse3(h, xh):
        o = (jnp.dot(h, sel3(4).astype(bf), preferred_element_type=jnp.float32)
             + jnp.dot(xh, sel3(5).astype(bf),
                       preferred_element_type=jnp.float32)
             + sel3(6))
        return _leaky(o)

    is_pre = (s < b0) | ((s >= b1) & (s < b2)) | ((s >= b3) & (s < b4))

    @pl.when(is_pre)
    def _prepass():
        r = jnp.where(s < b0, s, jnp.where(s < b2, s - b1, s - b3))
        rows = pl.ds(r * TILE, TILE)
        x0 = jnp.where(r < nut, mu_ref[...], mi_ref[...])
        nrm2 = jnp.sum(x0 * x0, axis=-1, keepdims=True)
        x0n = (x0 * jax.lax.rsqrt(jnp.maximum(nrm2, EPS * EPS))).astype(bf)
        xb = jnp.where(s < b0, x0n, x_ref[rows, :])

        y1 = jnp.dot(xb, sel3(0).astype(bf), preferred_element_type=jnp.float32)
        y1_ref[rows, :] = (y1 + sel3(1)).astype(bf)

        y2 = jnp.dot(xb, sel3(2).astype(bf), preferred_element_type=jnp.float32)
        xh_ref[rows, :] = (_leaky(y2 + sel3(3)) + id_ref[rows, :]).astype(bf)

    @pl.when((s >= b0) & (s < b1))
    def _l0_aggregate():
        t = s - b0
        rows = pl.ds(t * KROW, KROW)
        ab = a_ref[...].astype(bf)
        abf_ref[rows, :] = ab
        h = _leaky(jnp.dot(ab, y1_ref[...],
                           preferred_element_type=jnp.float32)).astype(bf)
        x_ref[rows, :] = fuse3(h, xh_ref[rows, :]).astype(bf)

    @pl.when(((s >= b2) & (s < b3)) | (s >= b4))
    def _l12_aggregate():
        i = jnp.where(s < b3, s - b2, s - b4)
        irows = pl.ds(i * TILE, TILE)
        acc = jnp.dot(abf_ref[irows, :], y1_ref[...],
                      preferred_element_type=jnp.float32)
        h = _leaky(acc).astype(bf)
        o = fuse3(h, xh_ref[irows, :])

        @pl.when(s < b3)
        def _():
            x_ref[irows, :] = o.astype(bf)

        @pl.when(s >= b4)
        def _():
            o_ref[...] = o


def kernel(mUser, m_item, id_embds, A,
           l0_w1, l0_b1, l0_w2, l0_b2, l0_w3h, l0_w3x, l0_b3,
           l1_w1, l1_b1, l1_w2, l1_b2, l1_w3h, l1_w3x, l1_b3,
           l2_w1, l2_b1, l2_w2, l2_b2, l2_w3h, l2_w3x, l2_b3):
    nu, d = mUser.shape
    ni = m_item.shape[0]
    n = nu + ni
    e = id_embds.shape[1]
    nt = n // TILE
    nut = nu // TILE
    nit = ni // TILE
    ntk = n // KROW
    b0 = nt
    b1 = b0 + ntk
    b2 = b1 + nt
    b3 = b2 + nt
    b4 = b3 + nt
    b5 = b4 + nt
    bnd = (b0, b1, b2, b3, b4, b5)

    zero2 = lambda s: (0, 0)
    mu_map = lambda s: (jnp.clip(s, 0, nut - 1), 0)
    mi_map = lambda s: (jnp.clip(s - nut, 0, nit - 1), 0)
    a_map = lambda s: (jnp.clip(s - b0, 0, ntk - 1), 0)
    o_map = lambda s: (jnp.clip(s - b4, 0, nt - 1), 0)

    weights = [l0_w1, l0_b1, l0_w2, l0_b2, l0_w3h, l0_w3x, l0_b3,
               l1_w1, l1_b1, l1_w2, l1_b2, l1_w3h, l1_w3x, l1_b3,
               l2_w1, l2_b1, l2_w2, l2_b2, l2_w3h, l2_w3x, l2_b3]

    body = functools.partial(_gcn_body, nut=nut, bnd=bnd)
    return pl.pallas_call(
        body,
        out_shape=jax.ShapeDtypeStruct((n, e), jnp.float32),
        grid=(b5,),
        in_specs=[
            pl.BlockSpec((TILE, d), mu_map),
            pl.BlockSpec((TILE, d), mi_map),
            pl.BlockSpec((n, e), zero2),
            pl.BlockSpec((KROW, n), a_map),
        ] + [
            pl.BlockSpec(wa.shape, zero2) for wa in weights
        ],
        out_specs=pl.BlockSpec((TILE, e), o_map),
        scratch_shapes=[
            pltpu.VMEM((n, n), jnp.bfloat16),
            pltpu.VMEM((n, d), jnp.bfloat16),
            pltpu.VMEM((n, e), jnp.bfloat16),
            pltpu.VMEM((n, e), jnp.bfloat16),
        ],
        compiler_params=pltpu.CompilerParams(
            dimension_semantics=("arbitrary",),
            vmem_limit_bytes=60 * 1024 * 1024),
    )(mUser, m_item, id_embds, A, *weights)


# speedup vs baseline: 4.7087x; 1.0720x over previous
import functools

import jax
import jax.numpy as jnp
from jax.experimental import pallas as pl
from jax.experimental.pallas import tpu as pltpu

NEG_SLOPE = 0.01
EPS = 1e-12
TILE = 512
KROW = 256


def _leaky(v):
    return jnp.where(v >= 0, v, NEG_SLOPE * v)


def _gcn_body(*refs, nut, bnd):
    b0, b1, b2, b3 = bnd
    mu_ref, mi_ref, id_ref, a_ref = refs[:4]
    w = refs[4:4 + 21]
    o_ref = refs[4 + 21]
    abf_ref, y1a_ref, y1b_ref, xha_ref, xhb_ref = refs[4 + 22:]

    s = pl.program_id(0)
    bf = jnp.bfloat16
    f32 = jnp.float32

    def prepass(xb, rows, y1_ref, xh_ref, li):
        y1 = jnp.dot(xb, w[7 * li][...].astype(bf), preferred_element_type=f32)
        y1_ref[rows, :] = (y1 + w[7 * li + 1][...]).astype(bf)
        y2 = jnp.dot(xb, w[7 * li + 2][...].astype(bf),
                     preferred_element_type=f32)
        xh_ref[rows, :] = (_leaky(y2 + w[7 * li + 3][...])
                           + id_ref[rows, :]).astype(bf)

    def fuse(h, xh, li):
        o = (jnp.dot(h, w[7 * li + 4][...].astype(bf),
                     preferred_element_type=f32)
             + jnp.dot(xh, w[7 * li + 5][...].astype(bf),
                       preferred_element_type=f32)
             + w[7 * li + 6][...])
        return _leaky(o)

    @pl.when(s < b0)
    def _l0_prepass():
        rows = pl.ds(s * TILE, TILE)
        x0 = jnp.where(s < nut, mu_ref[...], mi_ref[...])
        nrm2 = jnp.sum(x0 * x0, axis=-1, keepdims=True)
        x0n = (x0 * jax.lax.rsqrt(jnp.maximum(nrm2, EPS * EPS))).astype(bf)
        prepass(x0n, rows, y1a_ref, xha_ref, 0)

    @pl.when((s >= b0) & (s < b1))
    def _l0_aggregate():
        rows = pl.ds((s - b0) * KROW, KROW)
        ab = a_ref[...].astype(bf)
        abf_ref[rows, :] = ab
        h = _leaky(jnp.dot(ab, y1a_ref[...],
                           preferred_element_type=f32)).astype(bf)
        ob = fuse(h, xha_ref[rows, :], 0).astype(bf)
        prepass(ob, rows, y1b_ref, xhb_ref, 1)

    @pl.when((s >= b1) & (s < b2))
    def _l1_aggregate():
        irows = pl.ds((s - b1) * TILE, TILE)
        h = _leaky(jnp.dot(abf_ref[irows, :], y1b_ref[...],
                           preferred_element_type=f32)).astype(bf)
        ob = fuse(h, xhb_ref[irows, :], 1).astype(bf)
        prepass(ob, irows, y1a_ref, xha_ref, 2)

    @pl.when(s >= b2)
    def _l2_aggregate():
        irows = pl.ds((s - b2) * TILE, TILE)
        h = _leaky(jnp.dot(abf_ref[irows, :], y1a_ref[...],
                           preferred_element_type=f32)).astype(bf)
        o_ref[...] = fuse(h, xha_ref[irows, :], 2)


def kernel(mUser, m_item, id_embds, A,
           l0_w1, l0_b1, l0_w2, l0_b2, l0_w3h, l0_w3x, l0_b3,
           l1_w1, l1_b1, l1_w2, l1_b2, l1_w3h, l1_w3x, l1_b3,
           l2_w1, l2_b1, l2_w2, l2_b2, l2_w3h, l2_w3x, l2_b3):
    nu, d = mUser.shape
    ni = m_item.shape[0]
    n = nu + ni
    e = id_embds.shape[1]
    nt = n // TILE
    nut = nu // TILE
    nit = ni // TILE
    ntk = n // KROW
    b0 = nt
    b1 = b0 + ntk
    b2 = b1 + nt
    b3 = b2 + nt
    bnd = (b0, b1, b2, b3)

    zero2 = lambda s: (0, 0)
    mu_map = lambda s: (jnp.clip(s, 0, nut - 1), 0)
    mi_map = lambda s: (jnp.clip(s - nut, 0, nit - 1), 0)
    a_map = lambda s: (jnp.clip(s - b0, 0, ntk - 1), 0)
    o_map = lambda s: (jnp.clip(s - b2, 0, nt - 1), 0)

    weights = [l0_w1, l0_b1, l0_w2, l0_b2, l0_w3h, l0_w3x, l0_b3,
               l1_w1, l1_b1, l1_w2, l1_b2, l1_w3h, l1_w3x, l1_b3,
               l2_w1, l2_b1, l2_w2, l2_b2, l2_w3h, l2_w3x, l2_b3]

    body = functools.partial(_gcn_body, nut=nut, bnd=bnd)
    return pl.pallas_call(
        body,
        out_shape=jax.ShapeDtypeStruct((n, e), jnp.float32),
        grid=(b3,),
        in_specs=[
            pl.BlockSpec((TILE, d), mu_map),
            pl.BlockSpec((TILE, d), mi_map),
            pl.BlockSpec((n, e), zero2),
            pl.BlockSpec((KROW, n), a_map),
        ] + [
            pl.BlockSpec(wa.shape, zero2) for wa in weights
        ],
        out_specs=pl.BlockSpec((TILE, e), o_map),
        scratch_shapes=[
            pltpu.VMEM((n, n), jnp.bfloat16),
            pltpu.VMEM((n, d), jnp.bfloat16),
            pltpu.VMEM((n, d), jnp.bfloat16),
            pltpu.VMEM((n, e), jnp.bfloat16),
            pltpu.VMEM((n, e), jnp.bfloat16),
        ],
        compiler_params=pltpu.CompilerParams(
            dimension_semantics=("arbitrary",),
            vmem_limit_bytes=60 * 1024 * 1024),
    )(mUser, m_item, id_embds, A, *weights)


# speedup vs baseline: 5.0625x; 1.0751x over previous
import functools

import jax
import jax.numpy as jnp
from jax.experimental import pallas as pl
from jax.experimental.pallas import tpu as pltpu

NEG_SLOPE = 0.01
EPS = 1e-12
TILE = 512
KROW = 512


def _leaky(v):
    return jnp.where(v >= 0, v, NEG_SLOPE * v)


def _gcn_body(*refs, nut, bnd):
    b0, b1, b2, b3 = bnd
    mu_ref, mi_ref, id_ref, a_ref = refs[:4]
    w = refs[4:4 + 21]
    o_ref = refs[4 + 21]
    abf_ref, y1a_ref, y1b_ref, xha_ref, xhb_ref = refs[4 + 22:]

    s = pl.program_id(0)
    bf = jnp.bfloat16
    f32 = jnp.float32

    def prepass(xb, rows, y1_ref, xh_ref, li):
        y1 = jnp.dot(xb, w[7 * li][...].astype(bf), preferred_element_type=f32)
        y1_ref[rows, :] = (y1 + w[7 * li + 1][...]).astype(bf)
        y2 = jnp.dot(xb, w[7 * li + 2][...].astype(bf),
                     preferred_element_type=f32)
        xh_ref[rows, :] = (_leaky(y2 + w[7 * li + 3][...])
                           + id_ref[rows, :]).astype(bf)

    def fuse(h, xh, li):
        o = (jnp.dot(h, w[7 * li + 4][...].astype(bf),
                     preferred_element_type=f32)
             + jnp.dot(xh, w[7 * li + 5][...].astype(bf),
                       preferred_element_type=f32)
             + w[7 * li + 6][...])
        return _leaky(o)

    @pl.when(s < b0)
    def _l0_prepass():
        rows = pl.ds(s * TILE, TILE)
        x0 = jnp.where(s < nut, mu_ref[...], mi_ref[...])
        nrm2 = jnp.sum(x0 * x0, axis=-1, keepdims=True)
        x0n = (x0 * jax.lax.rsqrt(jnp.maximum(nrm2, EPS * EPS))).astype(bf)
        prepass(x0n, rows, y1a_ref, xha_ref, 0)

    @pl.when((s >= b0) & (s < b1))
    def _l0_aggregate():
        rows = pl.ds((s - b0) * KROW, KROW)
        ab = a_ref[...].astype(bf)
        abf_ref[rows, :] = ab
        h = _leaky(jnp.dot(ab, y1a_ref[...],
                           preferred_element_type=f32)).astype(bf)
        ob = fuse(h, xha_ref[rows, :], 0).astype(bf)
        prepass(ob, rows, y1b_ref, xhb_ref, 1)

    @pl.when((s >= b1) & (s < b2))
    def _l1_aggregate():
        irows = pl.ds((s - b1) * TILE, TILE)
        h = _leaky(jnp.dot(abf_ref[irows, :], y1b_ref[...],
                           preferred_element_type=f32)).astype(bf)
        ob = fuse(h, xhb_ref[irows, :], 1).astype(bf)
        prepass(ob, irows, y1a_ref, xha_ref, 2)

    @pl.when(s >= b2)
    def _l2_aggregate():
        irows = pl.ds((s - b2) * TILE, TILE)
        h = _leaky(jnp.dot(abf_ref[irows, :], y1a_ref[...],
                           preferred_element_type=f32)).astype(bf)
        o_ref[...] = fuse(h, xha_ref[irows, :], 2)


def kernel(mUser, m_item, id_embds, A,
           l0_w1, l0_b1, l0_w2, l0_b2, l0_w3h, l0_w3x, l0_b3,
           l1_w1, l1_b1, l1_w2, l1_b2, l1_w3h, l1_w3x, l1_b3,
           l2_w1, l2_b1, l2_w2, l2_b2, l2_w3h, l2_w3x, l2_b3):
    nu, d = mUser.shape
    ni = m_item.shape[0]
    n = nu + ni
    e = id_embds.shape[1]
    nt = n // TILE
    nut = nu // TILE
    nit = ni // TILE
    ntk = n // KROW
    b0 = nt
    b1 = b0 + ntk
    b2 = b1 + nt
    b3 = b2 + nt
    bnd = (b0, b1, b2, b3)

    zero2 = lambda s: (0, 0)
    mu_map = lambda s: (jnp.clip(s, 0, nut - 1), 0)
    mi_map = lambda s: (jnp.clip(s - nut, 0, nit - 1), 0)
    a_map = lambda s: (jnp.clip(s - b0, 0, ntk - 1), 0)
    o_map = lambda s: (jnp.clip(s - b2, 0, nt - 1), 0)

    weights = [l0_w1, l0_b1, l0_w2, l0_b2, l0_w3h, l0_w3x, l0_b3,
               l1_w1, l1_b1, l1_w2, l1_b2, l1_w3h, l1_w3x, l1_b3,
               l2_w1, l2_b1, l2_w2, l2_b2, l2_w3h, l2_w3x, l2_b3]

    body = functools.partial(_gcn_body, nut=nut, bnd=bnd)
    return pl.pallas_call(
        body,
        out_shape=jax.ShapeDtypeStruct((n, e), jnp.float32),
        grid=(b3,),
        in_specs=[
            pl.BlockSpec((TILE, d), mu_map),
            pl.BlockSpec((TILE, d), mi_map),
            pl.BlockSpec((n, e), zero2),
            pl.BlockSpec((KROW, n), a_map),
        ] + [
            pl.BlockSpec(wa.shape, zero2) for wa in weights
        ],
        out_specs=pl.BlockSpec((TILE, e), o_map),
        scratch_shapes=[
            pltpu.VMEM((n, n), jnp.bfloat16),
            pltpu.VMEM((n, d), jnp.bfloat16),
            pltpu.VMEM((n, d), jnp.bfloat16),
            pltpu.VMEM((n, e), jnp.bfloat16),
            pltpu.VMEM((n, e), jnp.bfloat16),
        ],
        compiler_params=pltpu.CompilerParams(
            dimension_semantics=("arbitrary",),
            vmem_limit_bytes=63 * 1024 * 1024),
    )(mUser, m_item, id_embds, A, *weights)


# speedup vs baseline: 5.4370x; 1.0740x over previous
import functools

import jax
import jax.numpy as jnp
from jax.experimental import pallas as pl
from jax.experimental.pallas import tpu as pltpu

NEG_SLOPE = 0.01
EPS = 1e-12
TILE = 512
KROW = 512
ATILE = 1024


def _leaky(v):
    return jnp.where(v >= 0, v, NEG_SLOPE * v)


def _gcn_body(*refs, nut, bnd):
    b0, b1, b2, b3 = bnd
    mu_ref, mi_ref, id_ref, a_ref = refs[:4]
    w = refs[4:4 + 21]
    o_ref = refs[4 + 21]
    abf_ref, y1a_ref, y1b_ref, xha_ref, xhb_ref = refs[4 + 22:]

    s = pl.program_id(0)
    bf = jnp.bfloat16
    f32 = jnp.float32

    def prepass(xb, rows, y1_ref, xh_ref, li):
        y1 = jnp.dot(xb, w[7 * li][...].astype(bf), preferred_element_type=f32)
        y1_ref[rows, :] = (y1 + w[7 * li + 1][...]).astype(bf)
        y2 = jnp.dot(xb, w[7 * li + 2][...].astype(bf),
                     preferred_element_type=f32)
        xh_ref[rows, :] = (_leaky(y2 + w[7 * li + 3][...])
                           + id_ref[rows, :]).astype(bf)

    def fuse(h, xh, li):
        o = (jnp.dot(h, w[7 * li + 4][...].astype(bf),
                     preferred_element_type=f32)
             + jnp.dot(xh, w[7 * li + 5][...].astype(bf),
                       preferred_element_type=f32)
             + w[7 * li + 6][...])
        return _leaky(o)

    @pl.when(s < b0)
    def _l0_prepass():
        rows = pl.ds(s * TILE, TILE)
        x0 = jnp.where(s < nut, mu_ref[...], mi_ref[...])
        nrm2 = jnp.sum(x0 * x0, axis=-1, keepdims=True)
        x0n = (x0 * jax.lax.rsqrt(jnp.maximum(nrm2, EPS * EPS))).astype(bf)
        prepass(x0n, rows, y1a_ref, xha_ref, 0)

    @pl.when((s >= b0) & (s < b1))
    def _l0_aggregate():
        rows = pl.ds((s - b0) * KROW, KROW)
        ab = a_ref[...].astype(bf)
        abf_ref[rows, :] = ab
        h = _leaky(jnp.dot(ab, y1a_ref[...],
                           preferred_element_type=f32)).astype(bf)
        ob = fuse(h, xha_ref[rows, :], 0).astype(bf)
        prepass(ob, rows, y1b_ref, xhb_ref, 1)

    @pl.when((s >= b1) & (s < b2))
    def _l1_aggregate():
        irows = pl.ds((s - b1) * ATILE, ATILE)
        h = _leaky(jnp.dot(abf_ref[irows, :], y1b_ref[...],
                           preferred_element_type=f32)).astype(bf)
        ob = fuse(h, xhb_ref[irows, :], 1).astype(bf)
        prepass(ob, irows, y1a_ref, xha_ref, 2)

    @pl.when(s >= b2)
    def _l2_aggregate():
        irows = pl.ds((s - b2) * ATILE, ATILE)
        h = _leaky(jnp.dot(abf_ref[irows, :], y1a_ref[...],
                           preferred_element_type=f32)).astype(bf)
        o_ref[...] = fuse(h, xha_ref[irows, :], 2)


def kernel(mUser, m_item, id_embds, A,
           l0_w1, l0_b1, l0_w2, l0_b2, l0_w3h, l0_w3x, l0_b3,
           l1_w1, l1_b1, l1_w2, l1_b2, l1_w3h, l1_w3x, l1_b3,
           l2_w1, l2_b1, l2_w2, l2_b2, l2_w3h, l2_w3x, l2_b3):
    nu, d = mUser.shape
    ni = m_item.shape[0]
    n = nu + ni
    e = id_embds.shape[1]
    nt = n // TILE
    nut = nu // TILE
    nit = ni // TILE
    ntk = n // KROW
    b0 = nt
    b1 = b0 + ntk
    nat = n // ATILE
    b2 = b1 + nat
    b3 = b2 + nat
    bnd = (b0, b1, b2, b3)

    zero2 = lambda s: (0, 0)
    mu_map = lambda s: (jnp.clip(s, 0, nut - 1), 0)
    mi_map = lambda s: (jnp.clip(s - nut, 0, nit - 1), 0)
    a_map = lambda s: (jnp.clip(s - b0, 0, ntk - 1), 0)
    o_map = lambda s: (jnp.clip(s - b2, 0, nat - 1), 0)

    weights = [l0_w1, l0_b1, l0_w2, l0_b2, l0_w3h, l0_w3x, l0_b3,
               l1_w1, l1_b1, l1_w2, l1_b2, l1_w3h, l1_w3x, l1_b3,
               l2_w1, l2_b1, l2_w2, l2_b2, l2_w3h, l2_w3x, l2_b3]

    body = functools.partial(_gcn_body, nut=nut, bnd=bnd)
    return pl.pallas_call(
        body,
        out_shape=jax.ShapeDtypeStruct((n, e), jnp.float32),
        grid=(b3,),
        in_specs=[
            pl.BlockSpec((TILE, d), mu_map),
            pl.BlockSpec((TILE, d), mi_map),
            pl.BlockSpec((n, e), zero2),
            pl.BlockSpec((KROW, n), a_map),
        ] + [
            pl.BlockSpec(wa.shape, zero2) for wa in weights
        ],
        out_specs=pl.BlockSpec((ATILE, e), o_map),
        scratch_shapes=[
            pltpu.VMEM((n, n), jnp.bfloat16),
            pltpu.VMEM((n, d), jnp.bfloat16),
            pltpu.VMEM((n, d), jnp.bfloat16),
            pltpu.VMEM((n, e), jnp.bfloat16),
            pltpu.VMEM((n, e), jnp.bfloat16),
        ],
        compiler_params=pltpu.CompilerParams(
            dimension_semantics=("arbitrary",),
            vmem_limit_bytes=63 * 1024 * 1024),
    )(mUser, m_item, id_embds, A, *weights)


# speedup vs baseline: 5.6423x; 1.0378x over previous
import functools

import jax
import jax.numpy as jnp
from jax.experimental import pallas as pl
from jax.experimental.pallas import tpu as pltpu

NEG_SLOPE = 0.01
EPS = 1e-12
TILE = 512
KROW = 512
ATILE = 2048


def _leaky(v):
    return jnp.where(v >= 0, v, NEG_SLOPE * v)


def _gcn_body(*refs, nut, bnd):
    b0, b1, b2, b3 = bnd
    mu_ref, mi_ref, id_ref, a_ref = refs[:4]
    w = refs[4:4 + 21]
    o_ref = refs[4 + 21]
    abf_ref, y1a_ref, y1b_ref, xha_ref, xhb_ref = refs[4 + 22:]

    s = pl.program_id(0)
    bf = jnp.bfloat16
    f32 = jnp.float32

    def prepass(xb, rows, y1_ref, xh_ref, li):
        y1 = jnp.dot(xb, w[7 * li][...].astype(bf), preferred_element_type=f32)
        y1_ref[rows, :] = (y1 + w[7 * li + 1][...]).astype(bf)
        y2 = jnp.dot(xb, w[7 * li + 2][...].astype(bf),
                     preferred_element_type=f32)
        xh_ref[rows, :] = (_leaky(y2 + w[7 * li + 3][...])
                           + id_ref[rows, :]).astype(bf)

    def fuse(h, xh, li):
        o = (jnp.dot(h, w[7 * li + 4][...].astype(bf),
                     preferred_element_type=f32)
             + jnp.dot(xh, w[7 * li + 5][...].astype(bf),
                       preferred_element_type=f32)
             + w[7 * li + 6][...])
        return _leaky(o)

    @pl.when(s < b0)
    def _l0_prepass():
        rows = pl.ds(s * TILE, TILE)
        x0 = jnp.where(s < nut, mu_ref[...], mi_ref[...])
        nrm2 = jnp.sum(x0 * x0, axis=-1, keepdims=True)
        x0n = (x0 * jax.lax.rsqrt(jnp.maximum(nrm2, EPS * EPS))).astype(bf)
        prepass(x0n, rows, y1a_ref, xha_ref, 0)

    @pl.when((s >= b0) & (s < b1))
    def _l0_aggregate():
        rows = pl.ds((s - b0) * KROW, KROW)
        ab = a_ref[...].astype(bf)
        abf_ref[rows, :] = ab
        h = _leaky(jnp.dot(ab, y1a_ref[...],
                           preferred_element_type=f32)).astype(bf)
        ob = fuse(h, xha_ref[rows, :], 0).astype(bf)
        prepass(ob, rows, y1b_ref, xhb_ref, 1)

    @pl.when((s >= b1) & (s < b2))
    def _l1_aggregate():
        irows = pl.ds((s - b1) * ATILE, ATILE)
        h = _leaky(jnp.dot(abf_ref[irows, :], y1b_ref[...],
                           preferred_element_type=f32)).astype(bf)
        ob = fuse(h, xhb_ref[irows, :], 1).astype(bf)
        prepass(ob, irows, y1a_ref, xha_ref, 2)

    @pl.when(s >= b2)
    def _l2_aggregate():
        irows = pl.ds((s - b2) * ATILE, ATILE)
        h = _leaky(jnp.dot(abf_ref[irows, :], y1a_ref[...],
                           preferred_element_type=f32)).astype(bf)
        o_ref[...] = fuse(h, xha_ref[irows, :], 2)


def kernel(mUser, m_item, id_embds, A,
           l0_w1, l0_b1, l0_w2, l0_b2, l0_w3h, l0_w3x, l0_b3,
           l1_w1, l1_b1, l1_w2, l1_b2, l1_w3h, l1_w3x, l1_b3,
           l2_w1, l2_b1, l2_w2, l2_b2, l2_w3h, l2_w3x, l2_b3):
    nu, d = mUser.shape
    ni = m_item.shape[0]
    n = nu + ni
    e = id_embds.shape[1]
    nt = n // TILE
    nut = nu // TILE
    nit = ni // TILE
    ntk = n // KROW
    b0 = nt
    b1 = b0 + ntk
    nat = n // ATILE
    b2 = b1 + nat
    b3 = b2 + nat
    bnd = (b0, b1, b2, b3)

    zero2 = lambda s: (0, 0)
    mu_map = lambda s: (jnp.clip(s, 0, nut - 1), 0)
    mi_map = lambda s: (jnp.clip(s - nut, 0, nit - 1), 0)
    a_map = lambda s: (jnp.clip(s - b0, 0, ntk - 1), 0)
    o_map = lambda s: (jnp.clip(s - b2, 0, nat - 1), 0)

    weights = [l0_w1, l0_b1, l0_w2, l0_b2, l0_w3h, l0_w3x, l0_b3,
               l1_w1, l1_b1, l1_w2, l1_b2, l1_w3h, l1_w3x, l1_b3,
               l2_w1, l2_b1, l2_w2, l2_b2, l2_w3h, l2_w3x, l2_b3]

    body = functools.partial(_gcn_body, nut=nut, bnd=bnd)
    return pl.pallas_call(
        body,
        out_shape=jax.ShapeDtypeStruct((n, e), jnp.float32),
        grid=(b3,),
        in_specs=[
            pl.BlockSpec((TILE, d), mu_map),
            pl.BlockSpec((TILE, d), mi_map),
            pl.BlockSpec((n, e), zero2),
            pl.BlockSpec((KROW, n), a_map),
        ] + [
            pl.BlockSpec(wa.shape, zero2) for wa in weights
        ],
        out_specs=pl.BlockSpec((ATILE, e), o_map),
        scratch_shapes=[
            pltpu.VMEM((n, n), jnp.bfloat16),
            pltpu.VMEM((n, d), jnp.bfloat16),
            pltpu.VMEM((n, d), jnp.bfloat16),
            pltpu.VMEM((n, e), jnp.bfloat16),
            pltpu.VMEM((n, e), jnp.bfloat16),
        ],
        compiler_params=pltpu.CompilerParams(
            dimension_semantics=("arbitrary",),
            vmem_limit_bytes=63 * 1024 * 1024),
    )(mUser, m_item, id_embds, A, *weights)
